# Initial kernel scaffold; baseline (speedup 1.0000x reference)
import functools

import jax
import jax.numpy as jnp
from jax import lax
from jax.experimental import pallas as pl
from jax.experimental.pallas import tpu as pltpu
from jax.experimental.pallas import tpu_sc as plsc

BATCH = 16384
F = 26
D = 16
NC = 2
NS = 16
NW = NC * NS
EPW = BATCH // NW
CH = 128
NCHUNK = EPW // CH
RPC = CH * F
PAT = 208

_mesh = plsc.VectorSubcoreMesh(core_axis_name="c", subcore_axis_name="s")


@functools.partial(
    pl.kernel,
    mesh=_mesh,
    out_type=jax.ShapeDtypeStruct((BATCH,), jnp.float32),
    scratch_types=[
        pltpu.VMEM((F, CH), jnp.int32),
        pltpu.VMEM((RPC, D), jnp.float32),
        pltpu.VMEM((EPW,), jnp.float32),
        pltpu.VMEM((PAT,), jnp.int32),
        pltpu.VMEM((16,), jnp.float32),
        pltpu.VMEM((16,), jnp.float32),
        pltpu.SemaphoreType.DMA,
    ],
)
def _emb_pool_kernel(x_hbm, pat_hbm, tbl_hbm, w_hbm, b_hbm, out_hbm,
                     xidx, rows, outb, pat_v, w_v, b_v, sem):
    wid = lax.axis_index("s") * NC + lax.axis_index("c")
    pltpu.sync_copy(pat_hbm, pat_v)
    pltpu.sync_copy(w_hbm, w_v)
    pltpu.sync_copy(b_hbm, b_v)
    wv = w_v[...]
    bv = b_v[...]
    lanes = lax.iota(jnp.int32, 16)

    for c in range(NCHUNK):
        rowbase = (wid * NCHUNK + c) * F
        pltpu.sync_copy(x_hbm.at[pl.ds(rowbase, F)], xidx)

        def idx_body(k, carry):
            row = k // 8
            col = (k % 8) * 16
            p = (k % 13) * 16
            xidx[row, pl.ds(col, 16)] = (
                xidx[row, pl.ds(col, 16)] + pat_v[pl.ds(p, 16)]
            )
            return carry

        lax.fori_loop(0, PAT, idx_body, 0)

        copies = []
        for j in range(F):
            copies.append(
                pltpu.async_copy(
                    tbl_hbm.at[xidx.at[j]], rows.at[pl.ds(j * CH, CH)], sem
                )
            )
        for cp in copies:
            cp.wait()

        def group_body(g, carry):
            def elem_body(i, vec):
                base = (g * 16 + i) * F
                acc = rows[base, :]
                for f in range(1, F):
                    acc = acc + rows[base + f, :]
                s = jnp.sum(acc * wv)
                return jnp.where(lanes == i, s, vec)

            vec = lax.fori_loop(0, 16, elem_body, jnp.zeros(16, jnp.float32))
            z = vec + bv
            y = 1.0 / (1.0 + jnp.exp(-z))
            outb[pl.ds(c * CH + g * 16, 16)] = y
            return carry

        lax.fori_loop(0, CH // 16, group_body, 0)

    pltpu.sync_copy(outb, out_hbm.at[pl.ds(wid * EPW, EPW)])


def kernel(x, offsets, emb_table, W, b):
    x2d = x.astype(jnp.int32).reshape(BATCH * F // CH, CH)
    pat = jnp.tile(offsets.astype(jnp.int32), PAT // F)
    wv = (W.reshape(D) * (1.0 / F)).astype(jnp.float32)
    b16 = jnp.broadcast_to(b.astype(jnp.float32), (16,))
    return _emb_pool_kernel(x2d, pat, emb_table, wv, b16)


# speedup vs baseline: 1.1618x; 1.1618x over previous
import functools

import jax
import jax.numpy as jnp
from jax import lax
from jax.experimental import pallas as pl
from jax.experimental.pallas import tpu as pltpu
from jax.experimental.pallas import tpu_sc as plsc

BATCH = 16384
F = 26
D = 16
NC = 2
NS = 16
NW = NC * NS
EPW = BATCH // NW
CH = 128
NCHUNK = EPW // CH
RPC = CH * F
PAT = 208

_mesh = plsc.VectorSubcoreMesh(core_axis_name="c", subcore_axis_name="s")


@functools.partial(
    pl.kernel,
    mesh=_mesh,
    out_type=jax.ShapeDtypeStruct((BATCH,), jnp.float32),
    compiler_params=pltpu.CompilerParams(use_tc_tiling_on_sc=False),
    scratch_types=[
        pltpu.VMEM((RPC,), jnp.int32),
        pltpu.VMEM((RPC, D), jnp.float32),
        pltpu.VMEM((EPW,), jnp.float32),
        pltpu.VMEM((PAT,), jnp.int32),
        pltpu.VMEM((16,), jnp.float32),
        pltpu.VMEM((16,), jnp.float32),
        pltpu.SemaphoreType.DMA,
    ],
)
def _emb_pool_kernel(x_hbm, pat_hbm, tbl_hbm, w_hbm, b_hbm, out_hbm,
                     xidx, rows, outb, pat_v, w_v, b_v, sem):
    wid = lax.axis_index("s") * NC + lax.axis_index("c")
    pltpu.sync_copy(pat_hbm, pat_v)
    pltpu.sync_copy(w_hbm, w_v)
    pltpu.sync_copy(b_hbm, b_v)
    wv = w_v[...]
    bv = b_v[...]
    lanes = lax.iota(jnp.int32, 16)
    perms = [lanes ^ 8, lanes ^ 4, lanes ^ 2, lanes ^ 1]
    _dnums = lax.GatherDimensionNumbers(
        offset_dims=(), collapsed_slice_dims=(0,), start_index_map=(0,)
    )

    def _shuffle(v, pm):
        return lax.gather(
            v, pm[:, None], _dnums, (1,),
            mode=lax.GatherScatterMode.PROMISE_IN_BOUNDS,
        )

    for c in range(NCHUNK):
        flatbase = (wid * NCHUNK + c) * RPC
        pltpu.sync_copy(x_hbm.at[pl.ds(flatbase, RPC)], xidx)

        def idx_body(k, carry):
            p = (k % 13) * 16
            xidx[pl.ds(k * 16, 16)] = xidx[pl.ds(k * 16, 16)] + pat_v[pl.ds(p, 16)]
            return carry

        lax.fori_loop(0, RPC // 16, idx_body, 0)

        copies = []
        for j in range(F):
            copies.append(
                pltpu.async_copy(
                    tbl_hbm.at[xidx.at[pl.ds(j * CH, CH)]],
                    rows.at[pl.ds(j * CH, CH)],
                    sem,
                )
            )
        for cp in copies:
            cp.wait()

        def group_body(g, carry):
            def elem_body(i, vec):
                base = (g * 16 + i) * F
                acc = rows[base, :]
                for f in range(1, F):
                    acc = acc + rows[base + f, :]
                t = acc * wv
                for pm in perms:
                    t = t + _shuffle(t, pm)
                return jnp.where(lanes == i, t, vec)

            vec = lax.fori_loop(0, 16, elem_body, jnp.zeros(16, jnp.float32))
            z = vec + bv
            y = 1.0 / (1.0 + jnp.exp(-z))
            outb[pl.ds(c * CH + g * 16, 16)] = y
            return carry

        lax.fori_loop(0, CH // 16, group_body, 0)

    pltpu.sync_copy(outb, out_hbm.at[pl.ds(wid * EPW, EPW)])


def kernel(x, offsets, emb_table, W, b):
    xflat = x.astype(jnp.int32).reshape(BATCH * F)
    pat = jnp.tile(offsets.astype(jnp.int32), PAT // F)
    wv = (W.reshape(D) * (1.0 / F)).astype(jnp.float32)
    b16 = jnp.broadcast_to(b.astype(jnp.float32), (16,))
    return _emb_pool_kernel(xflat, pat, emb_table, wv, b16)


# speedup vs baseline: 4.0512x; 3.4871x over previous
import functools

import jax
import jax.numpy as jnp
from jax import lax
from jax.experimental import pallas as pl
from jax.experimental.pallas import tpu as pltpu
from jax.experimental.pallas import tpu_sc as plsc

BATCH = 16384
F = 26
D = 16
V = 1000000
NC = 2
NS = 16
NW = NC * NS
EPW = BATCH // NW
QW = 4
TWBLK = 8192

_mesh = plsc.VectorSubcoreMesh(core_axis_name="c", subcore_axis_name="s")


def _tw_body(t_ref, w_ref, o_ref):
    o_ref[...] = jnp.sum(t_ref[...] * w_ref[...], axis=0)


_tw_call = pl.pallas_call(
    _tw_body,
    grid=(pl.cdiv(V, TWBLK),),
    in_specs=[
        pl.BlockSpec((D, TWBLK), lambda i: (0, i)),
        pl.BlockSpec((D, 1), lambda i: (0, 0)),
    ],
    out_specs=pl.BlockSpec((TWBLK,), lambda i: (i,)),
    out_shape=jax.ShapeDtypeStruct((V,), jnp.float32),
)


@functools.partial(
    pl.kernel,
    mesh=_mesh,
    out_type=jax.ShapeDtypeStruct((BATCH,), jnp.float32),
    compiler_params=pltpu.CompilerParams(use_tc_tiling_on_sc=False),
    scratch_types=[
        pltpu.VMEM((F * EPW,), jnp.int32),
        pltpu.VMEM((F * EPW,), jnp.float32),
        pltpu.VMEM((EPW,), jnp.float32),
        pltpu.VMEM((F + 16,), jnp.int32),
        pltpu.VMEM((16,), jnp.float32),
        pltpu.SemaphoreType.DMA,
    ],
)
def _sc_pool_kernel(xt_hbm, offs_hbm, tw_hbm, b_hbm, out_hbm,
                    xidx, twg, outb, offs_v, b_v, sem):
    wid = lax.axis_index("s") * NC + lax.axis_index("c")
    pltpu.sync_copy(offs_hbm, offs_v)
    pltpu.sync_copy(b_hbm, b_v)
    base = wid * EPW
    for f in range(F):
        pltpu.sync_copy(
            xt_hbm.at[pl.ds(f * BATCH + base, EPW)],
            xidx.at[pl.ds(f * EPW, EPW)],
        )

    def add_body(f, carry):
        off = offs_v[pl.ds(f, 16)][0]

        def vbody(v, c2):
            sl = pl.ds(f * EPW + v * 16, 16)
            xidx[sl] = xidx[sl] + off
            return c2

        lax.fori_loop(0, EPW // 16, vbody, 0)
        return carry

    lax.fori_loop(0, F, add_body, 0)

    waves = []
    for q in range(QW):
        cps = [
            pltpu.async_copy(
                tw_hbm.at[xidx.at[pl.ds(f * EPW + q * 128, 128)]],
                twg.at[pl.ds(f * EPW + q * 128, 128)],
                sem,
            )
            for f in range(F)
        ]
        if q:
            for cp in waves[q - 1]:
                cp.wait()
        waves.append(cps)
    for cp in waves[QW - 1]:
        cp.wait()

    bv = b_v[...]

    def pool_body(v, carry):
        acc = twg[pl.ds(v * 16, 16)]
        for f in range(1, F):
            acc = acc + twg[pl.ds(f * EPW + v * 16, 16)]
        z = acc + bv
        outb[pl.ds(v * 16, 16)] = 1.0 / (1.0 + jnp.exp(-z))
        return carry

    lax.fori_loop(0, EPW // 16, pool_body, 0)
    pltpu.sync_copy(outb, out_hbm.at[pl.ds(base, EPW)])


def kernel(x, offsets, emb_table, W, b):
    xt = x.astype(jnp.int32).T.reshape(BATCH * F)
    wv = (W * (1.0 / F)).astype(jnp.float32)
    tw = _tw_call(emb_table.T, wv)
    b16 = jnp.broadcast_to(b.astype(jnp.float32), (16,))
    offs48 = jnp.pad(offsets.astype(jnp.int32), (0, 16))
    return _sc_pool_kernel(xt, offs48, tw, b16)


# speedup vs baseline: 4.0867x; 1.0088x over previous
import functools

import jax
import jax.numpy as jnp
from jax import lax
from jax.experimental import pallas as pl
from jax.experimental.pallas import tpu as pltpu
from jax.experimental.pallas import tpu_sc as plsc

BATCH = 16384
F = 26
D = 16
V = 1000000
NC = 2
NS = 16
NW = NC * NS
EPW = BATCH // NW
QW = 4
TWBLK = 32768

_mesh = plsc.VectorSubcoreMesh(core_axis_name="c", subcore_axis_name="s")


def _tw_body(t_ref, w_ref, o_ref):
    o_ref[...] = jnp.sum(t_ref[...] * w_ref[...], axis=0, keepdims=True)


_tw_call = pl.pallas_call(
    _tw_body,
    grid=(pl.cdiv(V, TWBLK),),
    in_specs=[
        pl.BlockSpec((D, TWBLK), lambda i: (0, i)),
        pl.BlockSpec((D, 1), lambda i: (0, 0)),
    ],
    out_specs=pl.BlockSpec((1, TWBLK), lambda i: (0, i)),
    out_shape=jax.ShapeDtypeStruct((1, V), jnp.float32),
)


@functools.partial(
    pl.kernel,
    mesh=_mesh,
    out_type=jax.ShapeDtypeStruct((BATCH,), jnp.float32),
    compiler_params=pltpu.CompilerParams(use_tc_tiling_on_sc=False),
    scratch_types=[
        pltpu.VMEM((F * EPW,), jnp.int32),
        pltpu.VMEM((F * EPW,), jnp.float32),
        pltpu.VMEM((EPW,), jnp.float32),
        pltpu.VMEM((F + 16,), jnp.int32),
        pltpu.VMEM((16,), jnp.float32),
        pltpu.SemaphoreType.DMA,
    ],
)
def _sc_pool_kernel(xt_hbm, offs_hbm, tw_hbm, b_hbm, out_hbm,
                    xidx, twg, outb, offs_v, b_v, sem):
    wid = lax.axis_index("s") * NC + lax.axis_index("c")
    pltpu.sync_copy(offs_hbm, offs_v)
    pltpu.sync_copy(b_hbm, b_v)
    base = wid * EPW
    for f in range(F):
        pltpu.sync_copy(
            xt_hbm.at[pl.ds(f * BATCH + base, EPW)],
            xidx.at[pl.ds(f * EPW, EPW)],
        )

    def add_body(f, carry):
        off = offs_v[pl.ds(f, 16)][0]

        def vbody(v, c2):
            sl = pl.ds(f * EPW + v * 16, 16)
            xidx[sl] = xidx[sl] + off
            return c2

        lax.fori_loop(0, EPW // 16, vbody, 0)
        return carry

    lax.fori_loop(0, F, add_body, 0)

    waves = []
    for q in range(QW):
        cps = [
            pltpu.async_copy(
                tw_hbm.at[xidx.at[pl.ds(f * EPW + q * 128, 128)]],
                twg.at[pl.ds(f * EPW + q * 128, 128)],
                sem,
            )
            for f in range(F)
        ]
        if q:
            for cp in waves[q - 1]:
                cp.wait()
        waves.append(cps)
    for cp in waves[QW - 1]:
        cp.wait()

    bv = b_v[...]

    def pool_body(v, carry):
        acc = twg[pl.ds(v * 16, 16)]
        for f in range(1, F):
            acc = acc + twg[pl.ds(f * EPW + v * 16, 16)]
        z = acc + bv
        outb[pl.ds(v * 16, 16)] = 1.0 / (1.0 + jnp.exp(-z))
        return carry

    lax.fori_loop(0, EPW // 16, pool_body, 0)
    pltpu.sync_copy(outb, out_hbm.at[pl.ds(base, EPW)])


def kernel(x, offsets, emb_table, W, b):
    xt = x.astype(jnp.int32).T.reshape(BATCH * F)
    wv = (W * (1.0 / F)).astype(jnp.float32)
    tw = _tw_call(emb_table.T, wv).reshape(V)
    b16 = jnp.broadcast_to(b.astype(jnp.float32), (16,))
    offs48 = jnp.pad(offsets.astype(jnp.int32), (0, 16))
    return _sc_pool_kernel(xt, offs48, tw, b16)


# speedup vs baseline: 5.9159x; 1.4476x over previous
import functools

import jax
import jax.numpy as jnp
from jax import lax
from jax.experimental import pallas as pl
from jax.experimental.pallas import tpu as pltpu
from jax.experimental.pallas import tpu_sc as plsc

BATCH = 16384
F = 26
D = 16
V = 1000000
NC = 2
NS = 16
NW = NC * NS
EPW = BATCH // NW
QW = 4
TWBLK = 32768

_mesh = plsc.VectorSubcoreMesh(core_axis_name="c", subcore_axis_name="s")


def _tw_body(t_ref, w_ref, o_ref):
    o_ref[...] = jnp.sum(t_ref[...] * w_ref[...], axis=0)


_tw_call = pl.pallas_call(
    _tw_body,
    grid=(pl.cdiv(V, TWBLK),),
    in_specs=[
        pl.BlockSpec((D, TWBLK), lambda i: (0, i)),
        pl.BlockSpec((D, 1), lambda i: (0, 0)),
    ],
    out_specs=pl.BlockSpec((TWBLK,), lambda i: (i,)),
    out_shape=jax.ShapeDtypeStruct((V,), jnp.float32),
)


@functools.partial(
    pl.kernel,
    mesh=_mesh,
    out_type=jax.ShapeDtypeStruct((BATCH,), jnp.float32),
    compiler_params=pltpu.CompilerParams(use_tc_tiling_on_sc=False),
    scratch_types=[
        pltpu.VMEM((F * EPW,), jnp.int32),
        pltpu.VMEM((F * EPW,), jnp.float32),
        pltpu.VMEM((EPW,), jnp.float32),
        pltpu.VMEM((F + 16,), jnp.int32),
        pltpu.VMEM((16,), jnp.float32),
        pltpu.SemaphoreType.DMA,
    ],
)
def _sc_pool_kernel(xt_hbm, offs_hbm, tw_hbm, b_hbm, out_hbm,
                    xidx, twg, outb, offs_v, b_v, sem):
    wid = lax.axis_index("s") * NC + lax.axis_index("c")
    pltpu.sync_copy(offs_hbm, offs_v)
    pltpu.sync_copy(b_hbm, b_v)
    base = wid * EPW
    for f in range(F):
        pltpu.sync_copy(
            xt_hbm.at[pl.ds(f * BATCH + base, EPW)],
            xidx.at[pl.ds(f * EPW, EPW)],
        )

    def add_body(f, carry):
        off = offs_v[pl.ds(f, 16)][0]

        def vbody(v, c2):
            sl = pl.ds(f * EPW + v * 16, 16)
            xidx[sl] = xidx[sl] + off
            return c2

        lax.fori_loop(0, EPW // 16, vbody, 0)
        return carry

    lax.fori_loop(0, F, add_body, 0)

    waves = []
    for q in range(QW):
        cps = [
            pltpu.async_copy(
                tw_hbm.at[xidx.at[pl.ds(f * EPW + q * 128, 128)]],
                twg.at[pl.ds(f * EPW + q * 128, 128)],
                sem,
            )
            for f in range(F)
        ]
        if q:
            for cp in waves[q - 1]:
                cp.wait()
        waves.append(cps)
    for cp in waves[QW - 1]:
        cp.wait()

    bv = b_v[...]

    def pool_body(v, carry):
        acc = twg[pl.ds(v * 16, 16)]
        for f in range(1, F):
            acc = acc + twg[pl.ds(f * EPW + v * 16, 16)]
        z = acc + bv
        outb[pl.ds(v * 16, 16)] = 1.0 / (1.0 + jnp.exp(-z))
        return carry

    lax.fori_loop(0, EPW // 16, pool_body, 0)
    pltpu.sync_copy(outb, out_hbm.at[pl.ds(base, EPW)])


def kernel(x, offsets, emb_table, W, b):
    xt = x.astype(jnp.int32).T.reshape(BATCH * F)
    wv = (W * (1.0 / F)).astype(jnp.float32)
    tw = _tw_call(emb_table.T, wv)
    b16 = jnp.broadcast_to(b.astype(jnp.float32), (16,))
    offs48 = jnp.pad(offsets.astype(jnp.int32), (0, 16))
    return _sc_pool_kernel(xt, offs48, tw, b16)


# speedup vs baseline: 6.4458x; 1.0896x over previous
import functools

import jax
import jax.numpy as jnp
from jax import lax
from jax.experimental import pallas as pl
from jax.experimental.pallas import tpu as pltpu
from jax.experimental.pallas import tpu_sc as plsc

BATCH = 16384
F = 26
D = 16
V = 1000000
NC = 2
NS = 16
NW = NC * NS
EPW = BATCH // NW
QW = 4
TWBLK = 32768

_mesh = plsc.VectorSubcoreMesh(core_axis_name="c", subcore_axis_name="s")


def _tw_body(t_ref, w_ref, o_ref):
    o_ref[...] = jnp.sum(t_ref[...] * w_ref[...], axis=0)


_tw_call = pl.pallas_call(
    _tw_body,
    grid=(pl.cdiv(V, TWBLK),),
    in_specs=[
        pl.BlockSpec((D, TWBLK), lambda i: (0, i)),
        pl.BlockSpec((D, 1), lambda i: (0, 0)),
    ],
    out_specs=pl.BlockSpec((TWBLK,), lambda i: (i,)),
    out_shape=jax.ShapeDtypeStruct((V,), jnp.float32),
)


@functools.partial(
    pl.kernel,
    mesh=_mesh,
    out_type=jax.ShapeDtypeStruct((BATCH,), jnp.float32),
    compiler_params=pltpu.CompilerParams(use_tc_tiling_on_sc=False),
    scratch_types=[
        pltpu.VMEM((F * EPW,), jnp.int32),
        pltpu.VMEM((F * EPW,), jnp.float32),
        pltpu.VMEM((EPW,), jnp.float32),
        pltpu.VMEM((F + 16,), jnp.int32),
        pltpu.VMEM((16,), jnp.float32),
        pltpu.VMEM_SHARED((V,), jnp.float32),
        pltpu.SemaphoreType.DMA,
    ],
)
def _sc_pool_kernel(xt_hbm, offs_hbm, tw_hbm, b_hbm, out_hbm,
                    xidx, twg, outb, offs_v, b_v, tw_sp, sem):
    sid = lax.axis_index("s")
    wid = sid * NC + lax.axis_index("c")
    pltpu.sync_copy(offs_hbm, offs_v)
    pltpu.sync_copy(b_hbm, b_v)
    base = wid * EPW
    for f in range(F):
        pltpu.sync_copy(
            xt_hbm.at[pl.ds(f * BATCH + base, EPW)],
            xidx.at[pl.ds(f * EPW, EPW)],
        )

    def add_body(f, carry):
        off = offs_v[pl.ds(f, 16)][0]

        def vbody(v, c2):
            sl = pl.ds(f * EPW + v * 16, 16)
            xidx[sl] = xidx[sl] + off
            return c2

        lax.fori_loop(0, EPW // 16, vbody, 0)
        return carry

    lax.fori_loop(0, F, add_body, 0)

    @pl.when(sid < 8)
    def _():
        sl = pl.ds(sid * (V // 8), V // 8)
        pltpu.sync_copy(tw_hbm.at[sl], tw_sp.at[sl])

    plsc.subcore_barrier()

    waves = []
    for q in range(QW):
        cps = [
            pltpu.async_copy(
                tw_sp.at[xidx.at[pl.ds(f * EPW + q * 128, 128)]],
                twg.at[pl.ds(f * EPW + q * 128, 128)],
                sem,
            )
            for f in range(F)
        ]
        if q:
            for cp in waves[q - 1]:
                cp.wait()
        waves.append(cps)
    for cp in waves[QW - 1]:
        cp.wait()

    bv = b_v[...]

    def pool_body(v, carry):
        acc = twg[pl.ds(v * 16, 16)]
        for f in range(1, F):
            acc = acc + twg[pl.ds(f * EPW + v * 16, 16)]
        z = acc + bv
        outb[pl.ds(v * 16, 16)] = 1.0 / (1.0 + jnp.exp(-z))
        return carry

    lax.fori_loop(0, EPW // 16, pool_body, 0)
    pltpu.sync_copy(outb, out_hbm.at[pl.ds(base, EPW)])


def kernel(x, offsets, emb_table, W, b):
    xt = x.astype(jnp.int32).T.reshape(BATCH * F)
    wv = (W * (1.0 / F)).astype(jnp.float32)
    tw = _tw_call(emb_table.T, wv)
    b16 = jnp.broadcast_to(b.astype(jnp.float32), (16,))
    offs48 = jnp.pad(offsets.astype(jnp.int32), (0, 16))
    return _sc_pool_kernel(xt, offs48, tw, b16)


# speedup vs baseline: 8.3174x; 1.2904x over previous
import functools

import jax
import jax.numpy as jnp
from jax import lax
from jax.experimental import pallas as pl
from jax.experimental.pallas import tpu as pltpu
from jax.experimental.pallas import tpu_sc as plsc

BATCH = 16384
F = 26
D = 16
V = 1000000
NC = 2
NS = 16
NW = NC * NS
EPW = BATCH // NW
QW = 4
TWBLK = 65536

_mesh = plsc.VectorSubcoreMesh(core_axis_name="c", subcore_axis_name="s")


def _tw_body(t_ref, w_ref, o_ref):
    o_ref[...] = jnp.sum(t_ref[...] * w_ref[...], axis=0)


_tw_call = pl.pallas_call(
    _tw_body,
    grid=(pl.cdiv(V, TWBLK),),
    in_specs=[
        pl.BlockSpec((D, TWBLK), lambda i: (0, i)),
        pl.BlockSpec((D, 1), lambda i: (0, 0)),
    ],
    out_specs=pl.BlockSpec((TWBLK,), lambda i: (i,)),
    out_shape=jax.ShapeDtypeStruct((V,), jnp.float32),
)


@functools.partial(
    pl.kernel,
    mesh=_mesh,
    out_type=jax.ShapeDtypeStruct((BATCH,), jnp.float32),
    compiler_params=pltpu.CompilerParams(use_tc_tiling_on_sc=False),
    scratch_types=[
        pltpu.VMEM((F * EPW,), jnp.int32),
        pltpu.VMEM((F * EPW,), jnp.float32),
        pltpu.VMEM((EPW,), jnp.float32),
        pltpu.VMEM((F + 16,), jnp.int32),
        pltpu.VMEM((16,), jnp.float32),
        pltpu.VMEM_SHARED((V,), jnp.float32),
        pltpu.SemaphoreType.DMA,
    ],
)
def _sc_pool_kernel(xt_hbm, offs_hbm, tw_hbm, b_hbm, out_hbm,
                    xidx, twg, outb, offs_v, b_v, tw_sp, sem):
    sid = lax.axis_index("s")
    wid = sid * NC + lax.axis_index("c")
    pltpu.sync_copy(offs_hbm, offs_v)
    pltpu.sync_copy(b_hbm, b_v)
    base = wid * EPW

    def stage_body(f, carry):
        pltpu.async_copy(
            xt_hbm.at[pl.ds(f * BATCH + base, EPW)],
            xidx.at[pl.ds(f * EPW, EPW)],
            sem,
        )
        return carry

    lax.fori_loop(0, F, stage_body, 0)
    pltpu.make_async_copy(xt_hbm.at[pl.ds(0, F * EPW)], xidx, sem).wait()

    def add_body(f, carry):
        off = offs_v[pl.ds(f, 16)][0]

        def vbody(v, c2):
            sl = pl.ds(f * EPW + v * 16, 16)
            xidx[sl] = xidx[sl] + off
            return c2

        lax.fori_loop(0, EPW // 16, vbody, 0)
        return carry

    lax.fori_loop(0, F, add_body, 0)

    @pl.when(sid < 8)
    def _():
        sl = pl.ds(sid * (V // 8), V // 8)
        pltpu.sync_copy(tw_hbm.at[sl], tw_sp.at[sl])

    plsc.subcore_barrier()

    def gather_body(k, carry):
        sl = pl.ds(k * 128, 128)
        pltpu.async_copy(tw_sp.at[xidx.at[sl]], twg.at[sl], sem)
        return carry

    lax.fori_loop(0, F * EPW // 128, gather_body, 0)
    pltpu.make_async_copy(tw_hbm.at[pl.ds(0, F * EPW)], twg, sem).wait()

    bv = b_v[...]

    def pool_body(v, carry):
        acc = twg[pl.ds(v * 16, 16)]
        for f in range(1, F):
            acc = acc + twg[pl.ds(f * EPW + v * 16, 16)]
        z = acc + bv
        outb[pl.ds(v * 16, 16)] = 1.0 / (1.0 + jnp.exp(-z))
        return carry

    lax.fori_loop(0, EPW // 16, pool_body, 0)
    pltpu.sync_copy(outb, out_hbm.at[pl.ds(base, EPW)])


def kernel(x, offsets, emb_table, W, b):
    xt = x.astype(jnp.int32).T.reshape(BATCH * F)
    wv = (W * (1.0 / F)).astype(jnp.float32)
    tw = _tw_call(emb_table.T, wv)
    b16 = jnp.broadcast_to(b.astype(jnp.float32), (16,))
    offs48 = jnp.pad(offsets.astype(jnp.int32), (0, 16))
    return _sc_pool_kernel(xt, offs48, tw, b16)


# speedup vs baseline: 9.5645x; 1.1499x over previous
import functools

import jax
import jax.numpy as jnp
from jax import lax
from jax.experimental import pallas as pl
from jax.experimental.pallas import tpu as pltpu
from jax.experimental.pallas import tpu_sc as plsc

BATCH = 16384
F = 26
D = 16
V = 1000000
NC = 2
NS = 16
NW = NC * NS
EPW = BATCH // NW
QW = 4
TWBLK = 131072

_mesh = plsc.VectorSubcoreMesh(core_axis_name="c", subcore_axis_name="s")


def _tw_body(t_ref, w_ref, o_ref):
    o_ref[...] = jnp.sum(t_ref[...] * w_ref[...], axis=0)


_tw_call = pl.pallas_call(
    _tw_body,
    grid=(pl.cdiv(V, TWBLK),),
    in_specs=[
        pl.BlockSpec((D, TWBLK), lambda i: (0, i)),
        pl.BlockSpec((D, 1), lambda i: (0, 0)),
    ],
    out_specs=pl.BlockSpec((TWBLK,), lambda i: (i,)),
    out_shape=jax.ShapeDtypeStruct((V,), jnp.float32),
)


@functools.partial(
    pl.kernel,
    mesh=_mesh,
    out_type=jax.ShapeDtypeStruct((BATCH,), jnp.float32),
    compiler_params=pltpu.CompilerParams(use_tc_tiling_on_sc=False),
    scratch_types=[
        pltpu.VMEM((F * EPW,), jnp.int32),
        pltpu.VMEM((F * EPW,), jnp.float32),
        pltpu.VMEM((EPW,), jnp.float32),
        pltpu.VMEM((F + 16,), jnp.int32),
        pltpu.VMEM((16,), jnp.float32),
        pltpu.VMEM_SHARED((V,), jnp.float32),
        pltpu.SemaphoreType.DMA,
        pltpu.SemaphoreType.DMA,
    ],
)
def _sc_pool_kernel(xt_hbm, offs_hbm, tw_hbm, b_hbm, out_hbm,
                    xidx, twg, outb, offs_v, b_v, tw_sp, sem, sem_p):
    sid = lax.axis_index("s")
    wid = sid * NC + lax.axis_index("c")

    @pl.when(sid < 8)
    def _():
        sl = pl.ds(sid * (V // 8), V // 8)
        pltpu.async_copy(tw_hbm.at[sl], tw_sp.at[sl], sem_p)

    pltpu.sync_copy(offs_hbm, offs_v)
    pltpu.sync_copy(b_hbm, b_v)
    base = wid * EPW

    def stage_body(f, carry):
        pltpu.async_copy(
            xt_hbm.at[pl.ds(f * BATCH + base, EPW)],
            xidx.at[pl.ds(f * EPW, EPW)],
            sem,
        )
        return carry

    lax.fori_loop(0, F, stage_body, 0)
    pltpu.make_async_copy(xt_hbm.at[pl.ds(0, F * EPW)], xidx, sem).wait()

    def add_body(f, carry):
        off = offs_v[pl.ds(f, 16)][0]

        def vbody(v, c2):
            sl = pl.ds(f * EPW + v * 16, 16)
            xidx[sl] = xidx[sl] + off
            return c2

        lax.fori_loop(0, EPW // 16, vbody, 0)
        return carry

    lax.fori_loop(0, F, add_body, 0)

    @pl.when(sid < 8)
    def _():
        sl = pl.ds(sid * (V // 8), V // 8)
        pltpu.make_async_copy(tw_hbm.at[sl], tw_sp.at[sl], sem_p).wait()

    plsc.subcore_barrier()

    def gather_body(k, carry):
        sl = pl.ds(k * 128, 128)
        pltpu.async_copy(tw_sp.at[xidx.at[sl]], twg.at[sl], sem)
        return carry

    lax.fori_loop(0, F * EPW // 128, gather_body, 0)
    pltpu.make_async_copy(tw_hbm.at[pl.ds(0, F * EPW)], twg, sem).wait()

    bv = b_v[...]

    def pool_body(v, carry):
        acc = twg[pl.ds(v * 16, 16)]
        for f in range(1, F):
            acc = acc + twg[pl.ds(f * EPW + v * 16, 16)]
        z = acc + bv
        outb[pl.ds(v * 16, 16)] = 1.0 / (1.0 + jnp.exp(-z))
        return carry

    lax.fori_loop(0, EPW // 16, pool_body, 0)
    pltpu.sync_copy(outb, out_hbm.at[pl.ds(base, EPW)])


def kernel(x, offsets, emb_table, W, b):
    xt = x.astype(jnp.int32).T.reshape(BATCH * F)
    wv = (W * (1.0 / F)).astype(jnp.float32)
    tw = _tw_call(emb_table.T, wv)
    b16 = jnp.broadcast_to(b.astype(jnp.float32), (16,))
    offs48 = jnp.pad(offsets.astype(jnp.int32), (0, 16))
    return _sc_pool_kernel(xt, offs48, tw, b16)
